# Initial kernel scaffold; baseline (speedup 1.0000x reference)
import jax
import jax.numpy as jnp
from jax.experimental import pallas as pl


def kernel(x, W1, b1, g1, be1, W2, b2, g2, be2, W3, b3, g3, be3, Wo, bo):
    raise NotImplementedError("write your pallas kernel here")


# speedup vs baseline: 37.8933x; 37.8933x over previous
import functools

import jax
import jax.numpy as jnp
from jax import lax
from jax.experimental import pallas as pl


def _stencil_coeffs(t_len, dtype):
    t = lax.broadcasted_iota(jnp.int32, (t_len, 1), 0)
    inv_s2 = 0.7071067811865475
    inv_s3 = 0.5773502691896258

    def dis(s):
        edge = (s == 0) | (s == t_len - 1)
        return jnp.where(edge, inv_s2, inv_s3).astype(dtype)

    d0 = dis(t)
    c_self = d0 * d0
    c_prev = jnp.where(t >= 1, dis(t - 1), 0.0).astype(dtype) * d0
    c_next = jnp.where(t <= t_len - 2, dis(t + 1), 0.0).astype(dtype) * d0
    return c_self, c_prev, c_next


def _chain_conv(xw, c_self, c_prev, c_next):
    prev = jnp.roll(xw, 1, axis=0)
    nxt = jnp.roll(xw, -1, axis=0)
    return c_self * xw + c_prev * prev + c_next * nxt


def _layer_norm_relu(h, g, b, f):
    m = jnp.mean(h, axis=-1, keepdims=True)
    v = jnp.mean((h - m) * (h - m), axis=-1, keepdims=True)
    out = (h - m) * lax.rsqrt(v + 1e-5) * g + b
    return jnp.maximum(out, 0.0)


def _encoder_kernel(x_ref, w1_ref, b1_ref, g1_ref, be1_ref,
                    w2_ref, b2_ref, g2_ref, be2_ref,
                    w3_ref, b3_ref, g3_ref, be3_ref,
                    wo_ref, bo_ref, out_ref, *, t_len, bb):
    c_self, c_prev, c_next = _stencil_coeffs(t_len, jnp.float32)
    for i in range(bb):
        h = x_ref[i]
        h = jnp.dot(h, w1_ref[...], preferred_element_type=jnp.float32)
        h = _chain_conv(h, c_self, c_prev, c_next) + b1_ref[...]
        h = _layer_norm_relu(h, g1_ref[...], be1_ref[...], 12)

        h = jnp.dot(h, w2_ref[...], preferred_element_type=jnp.float32)
        h = _chain_conv(h, c_self, c_prev, c_next) + b2_ref[...]
        h = _layer_norm_relu(h, g2_ref[...], be2_ref[...], 12)

        h = jnp.dot(h, w3_ref[...], preferred_element_type=jnp.float32)
        h = _chain_conv(h, c_self, c_prev, c_next) + b3_ref[...]
        h = _layer_norm_relu(h, g3_ref[...], be3_ref[...], 24)

        out_ref[i] = (jnp.dot(h, wo_ref[...], preferred_element_type=jnp.float32)
                      + bo_ref[...])


@functools.partial(jax.jit, static_argnames=("interpret",))
def _run(x, w1t, b1, g1, be1, w2t, b2, g2, be2, w3t, b3, g3, be3, wot, bo,
         interpret=False):
    b_, t_, d_in = x.shape
    latent = wot.shape[1]
    bb = 4
    grid = (b_ // bb,)

    def xmap(i):
        return (i, 0, 0)

    def wmap(i):
        return (0, 0)

    small_specs = [
        pl.BlockSpec(w1t.shape, wmap),
        pl.BlockSpec(b1.shape, wmap),
        pl.BlockSpec(g1.shape, wmap),
        pl.BlockSpec(be1.shape, wmap),
        pl.BlockSpec(w2t.shape, wmap),
        pl.BlockSpec(b2.shape, wmap),
        pl.BlockSpec(g2.shape, wmap),
        pl.BlockSpec(be2.shape, wmap),
        pl.BlockSpec(w3t.shape, wmap),
        pl.BlockSpec(b3.shape, wmap),
        pl.BlockSpec(g3.shape, wmap),
        pl.BlockSpec(be3.shape, wmap),
        pl.BlockSpec(wot.shape, wmap),
        pl.BlockSpec(bo.shape, wmap),
    ]

    return pl.pallas_call(
        functools.partial(_encoder_kernel, t_len=t_, bb=bb),
        grid=grid,
        in_specs=[pl.BlockSpec((bb, t_, d_in), xmap)] + small_specs,
        out_specs=pl.BlockSpec((bb, t_, latent), xmap),
        out_shape=jax.ShapeDtypeStruct((b_, t_, latent), jnp.float32),
        interpret=interpret,
    )(x, w1t, b1, g1, be1, w2t, b2, g2, be2, w3t, b3, g3, be3, wot, bo)


def kernel(x, W1, b1, g1, be1, W2, b2, g2, be2, W3, b3, g3, be3, Wo, bo):
    return _run(x,
                W1.T, b1[None, :], g1[None, :], be1[None, :],
                W2.T, b2[None, :], g2[None, :], be2[None, :],
                W3.T, b3[None, :], g3[None, :], be3[None, :],
                Wo.T, bo[None, :])


# speedup vs baseline: 97.1051x; 2.5626x over previous
import functools

import jax
import jax.numpy as jnp
import numpy as np
from jax import lax
from jax.experimental import pallas as pl

_NB = 8


def _seg_avg_const(f, bw):
    blk = np.zeros((bw, bw), np.float32)
    blk[:f, :f] = 1.0 / f
    return np.kron(np.eye(_NB, dtype=np.float32), blk)


def _stencil_coeffs(t_len, dtype):
    t = lax.broadcasted_iota(jnp.int32, (t_len, 1), 0)
    inv_s2 = 0.7071067811865475
    inv_s3 = 0.5773502691896258

    def dis(s):
        edge = (s == 0) | (s == t_len - 1)
        return jnp.where(edge, inv_s2, inv_s3).astype(dtype)

    d0 = dis(t)
    c_self = d0 * d0
    c_prev = jnp.where(t >= 1, dis(t - 1), 0.0).astype(dtype) * d0
    c_next = jnp.where(t <= t_len - 2, dis(t + 1), 0.0).astype(dtype) * d0
    return c_self, c_prev, c_next


def _layer(h, m_ref, s_ref, ba_ref, g_ref, be_ref, c_self, c_prev, c_next):
    u = jnp.dot(h, m_ref[...], preferred_element_type=jnp.float32)
    hc = (c_self * u + c_prev * jnp.roll(u, 1, axis=0)
          + c_next * jnp.roll(u, -1, axis=0) + ba_ref[...])
    v = jnp.dot(hc * hc, s_ref[...], preferred_element_type=jnp.float32)
    return jnp.maximum(hc * lax.rsqrt(v + 1e-5) * g_ref[...] + be_ref[...],
                       0.0)


def _encoder_kernel(xp_ref,
                    m1_ref, s1_ref, ba1_ref, g1_ref, be1_ref,
                    m2_ref, s2_ref, ba2_ref, g2_ref, be2_ref,
                    m3_ref, s3_ref, ba3_ref, g3_ref, be3_ref,
                    wo_ref, bo_ref, out_ref, *, t_len, latent):
    c_self, c_prev, c_next = _stencil_coeffs(t_len, jnp.float32)
    h = xp_ref[0]
    h = _layer(h, m1_ref, s1_ref, ba1_ref, g1_ref, be1_ref,
               c_self, c_prev, c_next)
    h = _layer(h, m2_ref, s2_ref, ba2_ref, g2_ref, be2_ref,
               c_self, c_prev, c_next)
    h = _layer(h, m3_ref, s3_ref, ba3_ref, g3_ref, be3_ref,
               c_self, c_prev, c_next)
    oa = jnp.dot(h, wo_ref[...], preferred_element_type=jnp.float32)
    for s in range(_NB):
        out_ref[s] = oa[:, s * latent:(s + 1) * latent] + bo_ref[...]


def _blk_weight(wt, bw_in, bw_out):
    wt = wt - jnp.mean(wt, axis=1, keepdims=True)
    wt = jnp.pad(wt, ((0, bw_in - wt.shape[0]), (0, bw_out - wt.shape[1])))
    return jnp.kron(jnp.eye(_NB, dtype=wt.dtype), wt)


def _blk_vec(v, bw, center=False):
    if center:
        v = v - jnp.mean(v)
    return jnp.tile(jnp.pad(v, (0, bw - v.shape[0])), _NB)[None, :]


@functools.partial(jax.jit, static_argnames=("interpret",))
def _run(x, W1, b1, g1, be1, W2, b2, g2, be2, W3, b3, g3, be3, Wo, bo,
         interpret=False):
    b_, t_, d_in = x.shape
    latent = Wo.shape[0]
    nblk = b_ // _NB
    xp = x.reshape(nblk, _NB, t_, d_in).transpose(0, 2, 1, 3)
    xp = xp.reshape(nblk, t_, _NB * d_in)

    m1 = _blk_weight(W1.T, d_in, 16)
    m2 = _blk_weight(W2.T, 16, 16)
    m3 = _blk_weight(W3.T, 16, 32)
    s1 = jnp.asarray(_seg_avg_const(12, 16))
    s2 = s1
    s3 = jnp.asarray(_seg_avg_const(24, 32))
    ba1 = _blk_vec(b1, 16, center=True)
    ba2 = _blk_vec(b2, 16, center=True)
    ba3 = _blk_vec(b3, 32, center=True)
    g1b, be1b = _blk_vec(g1, 16), _blk_vec(be1, 16)
    g2b, be2b = _blk_vec(g2, 16), _blk_vec(be2, 16)
    g3b, be3b = _blk_vec(g3, 32), _blk_vec(be3, 32)
    wo_big = jnp.kron(jnp.eye(_NB, dtype=Wo.dtype),
                      jnp.pad(Wo.T, ((0, 8), (0, 0))))
    bo2 = bo[None, :]

    def xmap(i):
        return (i, 0, 0)

    def wmap(i):
        return (0, 0)

    params = [m1, s1, ba1, g1b, be1b,
              m2, s2, ba2, g2b, be2b,
              m3, s3, ba3, g3b, be3b,
              wo_big, bo2]
    param_specs = [pl.BlockSpec(p.shape, wmap) for p in params]

    return pl.pallas_call(
        functools.partial(_encoder_kernel, t_len=t_, latent=latent),
        grid=(nblk,),
        in_specs=[pl.BlockSpec((1, t_, _NB * d_in), xmap)] + param_specs,
        out_specs=pl.BlockSpec((_NB, t_, latent), xmap),
        out_shape=jax.ShapeDtypeStruct((b_, t_, latent), jnp.float32),
        interpret=interpret,
    )(xp, *params)


def kernel(x, W1, b1, g1, be1, W2, b2, g2, be2, W3, b3, g3, be3, Wo, bo):
    return _run(x, W1, b1, g1, be1, W2, b2, g2, be2, W3, b3, g3, be3, Wo, bo)
